# baseline (speedup 1.0000x reference)
import functools

import jax
import jax.numpy as jnp
from jax import lax
from jax.experimental import pallas as pl
from jax.experimental.pallas import tpu as pltpu
from jax.experimental.pallas import tpu_sc as plsc

_NC, _NS, _LANES = 2, 16, 16
_NW = _NC * _NS


@functools.partial(jax.jit, static_argnames=("B", "C", "E"))
def _sc_gather_mean(ctx_flat, emb, *, B, C, E):
    rpw = B // _NW
    nvec = E // _LANES
    mesh = plsc.VectorSubcoreMesh(
        core_axis_name="c", subcore_axis_name="s",
        num_cores=_NC, num_subcores=_NS)

    @functools.partial(
        pl.kernel,
        out_type=jax.ShapeDtypeStruct((B, E), jnp.float32),
        mesh=mesh,
        compiler_params=pltpu.CompilerParams(use_tc_tiling_on_sc=False),
        scratch_types=[
            pltpu.VMEM((2, rpw), jnp.int32),
            pltpu.VMEM((2, rpw, E), jnp.float32),
            pltpu.VMEM((rpw, E), jnp.float32),
            pltpu.SemaphoreType.DMA,
            pltpu.SemaphoreType.DMA,
        ],
    )
    def k(ctx_hbm, emb_hbm, out_hbm, idx_v, rows_v, acc_v, sem0, sem1):
        sems = (sem0, sem1)
        wid = lax.axis_index("s") * _NC + lax.axis_index("c")
        base = wid * rpw

        pltpu.sync_copy(ctx_hbm.at[pl.ds(base, rpw)], idx_v.at[0])
        pending = {0: pltpu.async_copy(emb_hbm.at[idx_v.at[0]], acc_v, sems[0])}
        if C > 1:
            pltpu.sync_copy(ctx_hbm.at[pl.ds(B + base, rpw)], idx_v.at[1])
            pending[1] = pltpu.async_copy(
                emb_hbm.at[idx_v.at[1]], rows_v.at[1], sems[1])
        pending.pop(0).wait()

        for j in range(1, C):
            b = j % 2
            if j + 1 < C:
                nb = (j + 1) % 2
                pltpu.sync_copy(
                    ctx_hbm.at[pl.ds((j + 1) * B + base, rpw)], idx_v.at[nb])
                pending[nb] = pltpu.async_copy(
                    emb_hbm.at[idx_v.at[nb]], rows_v.at[nb], sems[nb])
            pending.pop(b).wait()

            def accum(r, _, b=b):
                for c in range(nvec):
                    sl = pl.ds(c * _LANES, _LANES)
                    acc_v[r, sl] = acc_v[r, sl] + rows_v[b, r, sl]
                return 0
            lax.fori_loop(0, rpw, accum, 0)

        scale = jnp.float32(1.0 / C)

        def scale_row(r, _):
            for c in range(nvec):
                sl = pl.ds(c * _LANES, _LANES)
                acc_v[r, sl] = acc_v[r, sl] * scale
            return 0
        lax.fori_loop(0, rpw, scale_row, 0)
        pltpu.sync_copy(acc_v, out_hbm.at[pl.ds(base, rpw), :])

    return k(ctx_flat, emb)


def _tc_project_t(agg, w_t, out_b):
    B, E = agg.shape
    V = w_t.shape[1]
    BV = 512

    def body(w_ref, agg_ref, b_ref, o_ref):
        acc = lax.dot_general(
            w_ref[...], agg_ref[...],
            (((0,), (1,)), ((), ())),
            preferred_element_type=jnp.float32,
        )
        o_ref[...] = acc + lax.broadcast_in_dim(b_ref[...], (BV, B), (0,))

    return pl.pallas_call(
        body,
        grid=(pl.cdiv(V, BV),),
        in_specs=[
            pl.BlockSpec((E, BV), lambda i: (0, i)),
            pl.BlockSpec((B, E), lambda i: (0, 0)),
            pl.BlockSpec((BV,), lambda i: (i,)),
        ],
        out_specs=pl.BlockSpec((BV, B), lambda i: (i, 0)),
        out_shape=jax.ShapeDtypeStruct((V, B), jnp.float32),
    )(w_t, agg, out_b)


def kernel(context, embedding, out_w, out_b):
    B, C = context.shape
    V, E = embedding.shape
    ctx_flat = context.astype(jnp.int32).T.reshape(C * B)
    agg = _sc_gather_mean(ctx_flat, embedding, B=B, C=C, E=E)
    return _tc_project_t(agg, out_w.T, out_b).T


# speedup vs baseline: 1.0147x; 1.0147x over previous
import functools

import jax
import jax.numpy as jnp
from jax import lax
from jax.experimental import pallas as pl
from jax.experimental.pallas import tpu as pltpu
from jax.experimental.pallas import tpu_sc as plsc

_NC, _NS, _LANES = 2, 16, 16
_NW = _NC * _NS


@functools.partial(jax.jit, static_argnames=("B", "C", "E"))
def _sc_gather_mean(ctx_flat, emb, *, B, C, E):
    EP = emb.shape[1]
    rpw = B // _NW
    nvec = E // _LANES
    mesh = plsc.VectorSubcoreMesh(
        core_axis_name="c", subcore_axis_name="s",
        num_cores=_NC, num_subcores=_NS)

    @functools.partial(
        pl.kernel,
        out_type=jax.ShapeDtypeStruct((B, E), jnp.float32),
        mesh=mesh,
        compiler_params=pltpu.CompilerParams(use_tc_tiling_on_sc=False),
        scratch_types=[
            pltpu.VMEM((2, rpw), jnp.int32),
            pltpu.VMEM((2, rpw, EP), jnp.float32),
            pltpu.VMEM((rpw, EP), jnp.float32),
            pltpu.VMEM((rpw, E), jnp.float32),
            pltpu.SemaphoreType.DMA,
            pltpu.SemaphoreType.DMA,
        ],
    )
    def k(ctx_hbm, emb_hbm, out_hbm, idx_v, rows_v, acc_v, out_v, sem0, sem1):
        sems = (sem0, sem1)
        wid = lax.axis_index("s") * _NC + lax.axis_index("c")
        base = wid * rpw

        pltpu.sync_copy(ctx_hbm.at[pl.ds(base, rpw)], idx_v.at[0])
        pending = {0: pltpu.async_copy(emb_hbm.at[idx_v.at[0]], acc_v, sems[0])}
        if C > 1:
            pltpu.sync_copy(ctx_hbm.at[pl.ds(B + base, rpw)], idx_v.at[1])
            pending[1] = pltpu.async_copy(
                emb_hbm.at[idx_v.at[1]], rows_v.at[1], sems[1])
        pending.pop(0).wait()

        for j in range(1, C):
            b = j % 2
            if j + 1 < C:
                nb = (j + 1) % 2
                pltpu.sync_copy(
                    ctx_hbm.at[pl.ds((j + 1) * B + base, rpw)], idx_v.at[nb])
                pending[nb] = pltpu.async_copy(
                    emb_hbm.at[idx_v.at[nb]], rows_v.at[nb], sems[nb])
            pending.pop(b).wait()

            def accum(r, _, b=b):
                for c in range(nvec):
                    sl = pl.ds(c * _LANES, _LANES)
                    acc_v[r, sl] = acc_v[r, sl] + rows_v[b, r, sl]
                return 0
            lax.fori_loop(0, rpw, accum, 0)

        scale = jnp.float32(1.0 / C)

        def scale_row(r, _):
            for c in range(nvec):
                sl = pl.ds(c * _LANES, _LANES)
                out_v[r, sl] = acc_v[r, sl] * scale
            return 0
        lax.fori_loop(0, rpw, scale_row, 0)
        pltpu.sync_copy(out_v, out_hbm.at[pl.ds(base, rpw), :])

    return k(ctx_flat, emb)


def _tc_transpose_pad(emb_t):
    E, V = emb_t.shape
    BVT = 2048

    def body(x_ref, o_ref):
        xt = jnp.swapaxes(x_ref[...], 0, 1)
        o_ref[...] = jnp.concatenate(
            [xt, jnp.zeros((BVT, 128 - E), jnp.float32)], axis=1)

    return pl.pallas_call(
        body,
        grid=(pl.cdiv(V, BVT),),
        in_specs=[pl.BlockSpec((E, BVT), lambda i: (0, i))],
        out_specs=pl.BlockSpec((BVT, 128), lambda i: (i, 0)),
        out_shape=jax.ShapeDtypeStruct((V, 128), jnp.float32),
    )(emb_t)


def _tc_project_t(agg, w_t, out_b):
    B, E = agg.shape
    V = w_t.shape[1]
    BV = 512

    def body(w_ref, agg_ref, b_ref, o_ref):
        acc = lax.dot_general(
            w_ref[...], agg_ref[...],
            (((0,), (1,)), ((), ())),
            preferred_element_type=jnp.float32,
        )
        o_ref[...] = acc + lax.broadcast_in_dim(b_ref[...], (BV, B), (0,))

    return pl.pallas_call(
        body,
        grid=(pl.cdiv(V, BV),),
        in_specs=[
            pl.BlockSpec((E, BV), lambda i: (0, i)),
            pl.BlockSpec((B, E), lambda i: (0, 0)),
            pl.BlockSpec((BV,), lambda i: (i,)),
        ],
        out_specs=pl.BlockSpec((BV, B), lambda i: (i, 0)),
        out_shape=jax.ShapeDtypeStruct((V, B), jnp.float32),
    )(w_t, agg, out_b)


def kernel(context, embedding, out_w, out_b):
    B, C = context.shape
    V, E = embedding.shape
    ctx_flat = context.astype(jnp.int32).T.reshape(C * B)
    emb128 = _tc_transpose_pad(embedding.T)
    agg = _sc_gather_mean(ctx_flat, emb128, B=B, C=C, E=E)
    return _tc_project_t(agg, out_w.T, out_b).T


# speedup vs baseline: 1.0509x; 1.0356x over previous
import functools

import jax
import jax.numpy as jnp
from jax import lax
from jax.experimental import pallas as pl
from jax.experimental.pallas import tpu as pltpu
from jax.experimental.pallas import tpu_sc as plsc

_NC, _NS, _LANES = 2, 16, 16
_NW = _NC * _NS


@functools.partial(jax.jit, static_argnames=("B", "C", "E"))
def _sc_gather_mean(ctx_flat, emb, *, B, C, E):
    EP = emb.shape[1]
    rpw = B // _NW
    nvec = E // _LANES
    mesh = plsc.VectorSubcoreMesh(
        core_axis_name="c", subcore_axis_name="s",
        num_cores=_NC, num_subcores=_NS)

    @functools.partial(
        pl.kernel,
        out_type=jax.ShapeDtypeStruct((B, E), jnp.float32),
        mesh=mesh,
        compiler_params=pltpu.CompilerParams(use_tc_tiling_on_sc=False),
        scratch_types=[
            pltpu.VMEM((2, rpw), jnp.int32),
            pltpu.VMEM((2, rpw, EP), jnp.float32),
            pltpu.VMEM((rpw, EP), jnp.float32),
            pltpu.VMEM((rpw, E), jnp.float32),
            pltpu.SemaphoreType.DMA,
            pltpu.SemaphoreType.DMA,
        ],
    )
    def k(ctx_hbm, emb_hbm, out_hbm, idx_v, rows_v, acc_v, out_v, sem0, sem1):
        sems = (sem0, sem1)
        wid = lax.axis_index("s") * _NC + lax.axis_index("c")
        base = wid * rpw

        pltpu.sync_copy(ctx_hbm.at[pl.ds(base, rpw)], idx_v.at[0])
        pending = {0: pltpu.async_copy(emb_hbm.at[idx_v.at[0]], acc_v, sems[0])}
        if C > 1:
            pltpu.sync_copy(ctx_hbm.at[pl.ds(B + base, rpw)], idx_v.at[1])
            pending[1] = pltpu.async_copy(
                emb_hbm.at[idx_v.at[1]], rows_v.at[1], sems[1])
        pending.pop(0).wait()

        for j in range(1, C):
            b = j % 2
            if j + 1 < C:
                nb = (j + 1) % 2
                pltpu.sync_copy(
                    ctx_hbm.at[pl.ds((j + 1) * B + base, rpw)], idx_v.at[nb])
                pending[nb] = pltpu.async_copy(
                    emb_hbm.at[idx_v.at[nb]], rows_v.at[nb], sems[nb])
            pending.pop(b).wait()

            def accum(r, _, b=b):
                for c in range(nvec):
                    sl = pl.ds(c * _LANES, _LANES)
                    acc_v[r, sl] = acc_v[r, sl] + rows_v[b, r, sl]
                return 0
            lax.fori_loop(0, rpw, accum, 0)

        scale = jnp.float32(1.0 / C)

        def scale_row(r, _):
            for c in range(nvec):
                sl = pl.ds(c * _LANES, _LANES)
                out_v[r, sl] = acc_v[r, sl] * scale
            return 0
        lax.fori_loop(0, rpw, scale_row, 0)
        pltpu.sync_copy(out_v, out_hbm.at[pl.ds(base, rpw), :])

    return k(ctx_flat, emb)


def _tc_transpose_pad(emb_t):
    E, V = emb_t.shape
    BVT = 4096

    def body(x_ref, o_ref):
        xt = jnp.swapaxes(x_ref[...], 0, 1)
        o_ref[...] = jnp.concatenate(
            [xt, jnp.zeros((BVT, 128 - E), jnp.float32)], axis=1)

    return pl.pallas_call(
        body,
        grid=(pl.cdiv(V, BVT),),
        in_specs=[pl.BlockSpec((E, BVT), lambda i: (0, i))],
        out_specs=pl.BlockSpec((BVT, 128), lambda i: (i, 0)),
        out_shape=jax.ShapeDtypeStruct((V, 128), jnp.float32),
    )(emb_t)


def _tc_project_t(agg, w_t, out_b):
    B, E = agg.shape
    V = w_t.shape[1]
    BV = 1024

    def body(w_ref, agg_ref, b_ref, o_ref):
        acc = lax.dot_general(
            w_ref[...], agg_ref[...],
            (((0,), (1,)), ((), ())),
            preferred_element_type=jnp.float32,
        )
        o_ref[...] = acc + lax.broadcast_in_dim(b_ref[...], (BV, B), (0,))

    return pl.pallas_call(
        body,
        grid=(pl.cdiv(V, BV),),
        in_specs=[
            pl.BlockSpec((E, BV), lambda i: (0, i)),
            pl.BlockSpec((B, E), lambda i: (0, 0)),
            pl.BlockSpec((BV,), lambda i: (i,)),
        ],
        out_specs=pl.BlockSpec((BV, B), lambda i: (i, 0)),
        out_shape=jax.ShapeDtypeStruct((V, B), jnp.float32),
    )(w_t, agg, out_b)


def kernel(context, embedding, out_w, out_b):
    B, C = context.shape
    V, E = embedding.shape
    ctx_flat = context.astype(jnp.int32).T.reshape(C * B)
    emb128 = _tc_transpose_pad(embedding.T)
    agg = _sc_gather_mean(ctx_flat, emb128, B=B, C=C, E=E)
    return _tc_project_t(agg, out_w.T, out_b).T


# speedup vs baseline: 1.0510x; 1.0001x over previous
import functools

import jax
import jax.numpy as jnp
from jax import lax
from jax.experimental import pallas as pl
from jax.experimental.pallas import tpu as pltpu
from jax.experimental.pallas import tpu_sc as plsc

_NC, _NS, _LANES = 2, 16, 16
_NW = _NC * _NS


@functools.partial(jax.jit, static_argnames=("B", "C", "E"))
def _sc_gather_mean(ctx_flat, emb, *, B, C, E):
    EP = emb.shape[1]
    rpw = B // _NW
    nvec = E // _LANES
    mesh = plsc.VectorSubcoreMesh(
        core_axis_name="c", subcore_axis_name="s",
        num_cores=_NC, num_subcores=_NS)

    @functools.partial(
        pl.kernel,
        out_type=jax.ShapeDtypeStruct((B, EP), jnp.float32),
        mesh=mesh,
        compiler_params=pltpu.CompilerParams(use_tc_tiling_on_sc=False),
        scratch_types=[
            pltpu.VMEM((2, rpw), jnp.int32),
            pltpu.VMEM((2, rpw, EP), jnp.float32),
            pltpu.VMEM((rpw, EP), jnp.float32),
            pltpu.SemaphoreType.DMA,
            pltpu.SemaphoreType.DMA,
        ],
    )
    def k(ctx_hbm, emb_hbm, out_hbm, idx_v, rows_v, acc_v, sem0, sem1):
        sems = (sem0, sem1)
        wid = lax.axis_index("s") * _NC + lax.axis_index("c")
        base = wid * rpw

        pltpu.sync_copy(ctx_hbm.at[pl.ds(base, rpw)], idx_v.at[0])
        pending = {0: pltpu.async_copy(emb_hbm.at[idx_v.at[0]], acc_v, sems[0])}
        if C > 1:
            pltpu.sync_copy(ctx_hbm.at[pl.ds(B + base, rpw)], idx_v.at[1])
            pending[1] = pltpu.async_copy(
                emb_hbm.at[idx_v.at[1]], rows_v.at[1], sems[1])
        pending.pop(0).wait()

        for j in range(1, C):
            b = j % 2
            if j + 1 < C:
                nb = (j + 1) % 2
                pltpu.sync_copy(
                    ctx_hbm.at[pl.ds((j + 1) * B + base, rpw)], idx_v.at[nb])
                pending[nb] = pltpu.async_copy(
                    emb_hbm.at[idx_v.at[nb]], rows_v.at[nb], sems[nb])
            pending.pop(b).wait()

            def accum(r, _, b=b):
                for c in range(nvec):
                    sl = pl.ds(c * _LANES, _LANES)
                    acc_v[r, sl] = acc_v[r, sl] + rows_v[b, r, sl]
                return 0
            lax.fori_loop(0, rpw, accum, 0)

        pltpu.sync_copy(acc_v, out_hbm.at[pl.ds(base, rpw), :])

    return k(ctx_flat, emb)


def _tc_transpose_pad(emb_t):
    E, V = emb_t.shape
    BVT = 4096

    def body(x_ref, o_ref):
        xt = jnp.swapaxes(x_ref[...], 0, 1)
        o_ref[...] = jnp.concatenate(
            [xt, jnp.zeros((BVT, 128 - E), jnp.float32)], axis=1)

    return pl.pallas_call(
        body,
        grid=(pl.cdiv(V, BVT),),
        in_specs=[pl.BlockSpec((E, BVT), lambda i: (0, i))],
        out_specs=pl.BlockSpec((BVT, 128), lambda i: (i, 0)),
        out_shape=jax.ShapeDtypeStruct((V, 128), jnp.float32),
    )(emb_t)


def _tc_project_t(agg_sum, w_t, out_b, scale):
    B, EP = agg_sum.shape
    E, V = w_t.shape
    BV = 1024
    fscale = float(scale)

    def body(w_ref, agg_ref, b_ref, o_ref):
        acc = lax.dot_general(
            w_ref[...], agg_ref[...][:, :E],
            (((0,), (1,)), ((), ())),
            preferred_element_type=jnp.float32,
        )
        o_ref[...] = acc * fscale + lax.broadcast_in_dim(b_ref[...], (BV, B), (0,))

    return pl.pallas_call(
        body,
        grid=(pl.cdiv(V, BV),),
        in_specs=[
            pl.BlockSpec((E, BV), lambda i: (0, i)),
            pl.BlockSpec((B, EP), lambda i: (0, 0)),
            pl.BlockSpec((BV,), lambda i: (i,)),
        ],
        out_specs=pl.BlockSpec((BV, B), lambda i: (i, 0)),
        out_shape=jax.ShapeDtypeStruct((V, B), jnp.float32),
    )(w_t, agg_sum, out_b)


def kernel(context, embedding, out_w, out_b):
    B, C = context.shape
    V, E = embedding.shape
    ctx_flat = context.astype(jnp.int32).T.reshape(C * B)
    emb128 = _tc_transpose_pad(embedding.T)
    agg_sum = _sc_gather_mean(ctx_flat, emb128, B=B, C=C, E=E)
    return _tc_project_t(agg_sum, out_w.T, out_b, 1.0 / C).T
